# Initial kernel scaffold; baseline (speedup 1.0000x reference)
import jax
import jax.numpy as jnp
from jax.experimental import pallas as pl


def kernel(coords, idx, shape_code):
    raise NotImplementedError("write your pallas kernel here")


# speedup vs baseline: 13.2771x; 13.2771x over previous
import functools

import jax
import jax.numpy as jnp
from jax import lax
from jax.experimental import pallas as pl
from jax.experimental.pallas import tpu as pltpu
from jax.experimental.pallas import tpu_sc as plsc

B = 8
P = 20000
F = 3
C = 64
CODE_NUM = 256
COLS = CODE_NUM * F
TPAD = C + 1
NW = 32
PPW = P * B // NW
NCH = 25
CH = PPW // NCH
CB = CH * F + 24
NG = (CH * F + 8) // 16


def _sc_body(coords_hbm, idxp_hbm, code_hbm, out_hbm,
             idx_v, slice_v, t_v, coords_v, col_v, frac_v, out_v):
    cid = lax.axis_index("c")
    sid = lax.axis_index("s")
    wid = sid * 2 + cid
    b = wid // 4
    q = wid - b * 4

    lanes = lax.iota(jnp.int32, 16)

    pltpu.sync_copy(idxp_hbm, idx_v)
    myidx = plsc.load_gather(idx_v, [jnp.zeros((16,), jnp.int32) + b])[0]
    off = myidx * COLS
    pltpu.sync_copy(code_hbm.at[:, pl.ds(off, COLS)], slice_v)

    def tr_body(j, _):
        base = (j * 16 + lanes) * TPAD
        for c in range(C):
            v = slice_v[c, pl.ds(j * 16, 16)]
            plsc.store_scatter(t_v, [base + c], v)
        return 0
    lax.fori_loop(0, COLS // 16, tr_body, 0)

    def chunk_body(i, _):
        p0 = q * PPW + i * CH
        pltpu.sync_copy(coords_hbm.at[pl.ds((b * P + p0) * F, CH * F)],
                        coords_v.at[pl.ds(0, CH * F)])

        def pa_body(g, _):
            e0 = g * 16
            v = coords_v[pl.ds(e0, 16)]
            sc = (v + 1.0) * ((CODE_NUM - 1) / 2.0)
            c0 = sc.astype(jnp.int32)
            frac = sc - c0.astype(jnp.float32)
            f_lane = lax.rem(e0 + lanes, F)
            col = c0 + CODE_NUM * f_lane
            col_v[pl.ds(e0, 16)] = col
            frac_v[pl.ds(e0, 16)] = frac
            return 0
        lax.fori_loop(0, NG, pa_body, 0)

        def pb_body(p, _):
            p3 = p * F
            po = p * C
            colv = col_v[pl.ds(p3, 16)]
            fracv = frac_v[pl.ds(p3, 16)]
            accs = [None] * 4
            for f in range(F):
                cf = colv[f]
                tf = fracv[f]
                sf = 1.0 - tf
                base = cf * TPAD
                for k in range(4):
                    v0 = t_v[pl.ds(base + 16 * k, 16)]
                    v1 = t_v[pl.ds(base + TPAD + 16 * k, 16)]
                    term = v0 * sf + v1 * tf
                    accs[k] = term if accs[k] is None else accs[k] + term
            for k in range(4):
                out_v[pl.ds(po + 16 * k, 16)] = accs[k]
            return 0
        lax.fori_loop(0, CH, pb_body, 0)

        pltpu.sync_copy(out_v, out_hbm.at[pl.ds((b * P + p0) * C, CH * C)])
        return 0
    lax.fori_loop(0, NCH, chunk_body, 0)


@functools.partial(jax.jit, static_argnames=())
def kernel(coords, idx, shape_code):
    mesh = plsc.VectorSubcoreMesh(core_axis_name="c", subcore_axis_name="s")
    run = pl.kernel(
        _sc_body,
        out_type=jax.ShapeDtypeStruct((B * P * C,), jnp.float32),
        mesh=mesh,
        scratch_types=[
            pltpu.VMEM((16,), jnp.int32),
            pltpu.VMEM((C, COLS), jnp.float32),
            pltpu.VMEM((COLS * TPAD,), jnp.float32),
            pltpu.VMEM((CB,), jnp.float32),
            pltpu.VMEM((CB,), jnp.int32),
            pltpu.VMEM((CB,), jnp.float32),
            pltpu.VMEM((CH * C,), jnp.float32),
        ],
        compiler_params=pltpu.CompilerParams(needs_layout_passes=False),
    )
    coords2 = coords.reshape(B * P * F)
    idxp = jnp.pad(idx.astype(jnp.int32), (0, 16 - B))
    out = run(coords2, idxp, shape_code)
    return out.reshape(B, P, C)


# speedup vs baseline: 13.2803x; 1.0002x over previous
import functools

import jax
import jax.numpy as jnp
from jax import lax
from jax.experimental import pallas as pl
from jax.experimental.pallas import tpu as pltpu
from jax.experimental.pallas import tpu_sc as plsc

B = 8
P = 20000
F = 3
C = 64
CODE_NUM = 256
COLS = CODE_NUM * F
TPAD = C + 1
NW = 32
PPW = P * B // NW
NCH = 25
CH = PPW // NCH
CB = CH * F + 24
NG = (CH * F + 8) // 16


def _sc_body(coords_hbm, idxp_hbm, code_hbm, out_hbm,
             idx_v, slice_v, t_v, coords_v, col_v, frac_v, out_v):
    cid = lax.axis_index("c")
    sid = lax.axis_index("s")
    wid = sid * 2 + cid
    b = wid // 4
    q = wid - b * 4

    lanes = lax.iota(jnp.int32, 16)

    pltpu.sync_copy(idxp_hbm, idx_v)
    myidx = plsc.load_gather(idx_v, [jnp.zeros((16,), jnp.int32) + b])[0]
    off = myidx * COLS
    pltpu.sync_copy(code_hbm.at[:, pl.ds(off, COLS)], slice_v)

    def tr_body(j, _):
        base = (j * 16 + lanes) * TPAD
        for c in range(C):
            v = slice_v[c, pl.ds(j * 16, 16)]
            plsc.store_scatter(t_v, [base + c], v)
        return 0
    lax.fori_loop(0, COLS // 16, tr_body, 0)

    def chunk_body(i, _):
        p0 = q * PPW + i * CH
        pltpu.sync_copy(coords_hbm.at[pl.ds((b * P + p0) * F, CH * F)],
                        coords_v.at[pl.ds(0, CH * F)])

        def pa_body(g, _):
            e0 = g * 16
            v = coords_v[pl.ds(e0, 16)]
            sc = (v + 1.0) * ((CODE_NUM - 1) / 2.0)
            c0 = sc.astype(jnp.int32)
            frac = sc - c0.astype(jnp.float32)
            f_lane = lax.rem(e0 + lanes, F)
            col = c0 + CODE_NUM * f_lane
            col_v[pl.ds(e0, 16)] = col
            frac_v[pl.ds(e0, 16)] = frac
            return 0
        lax.fori_loop(0, NG, pa_body, 0)

        def pb_body(p, _):
            p3 = p * F
            po = p * C
            colv = col_v[pl.ds(p3, 16)]
            fracv = frac_v[pl.ds(p3, 16)]
            accs = [None] * 4
            for f in range(F):
                cf = colv[f]
                tf = fracv[f]
                sf = 1.0 - tf
                base = cf * TPAD
                for k in range(4):
                    v0 = t_v[pl.ds(base + 16 * k, 16)]
                    v1 = t_v[pl.ds(base + TPAD + 16 * k, 16)]
                    term = v0 * sf + v1 * tf
                    accs[k] = term if accs[k] is None else accs[k] + term
            for k in range(4):
                out_v[pl.ds(po + 16 * k, 16)] = accs[k]
            return 0
        lax.fori_loop(0, CH, pb_body, 0)

        pltpu.sync_copy(out_v, out_hbm.at[pl.ds((b * P + p0) * C, CH * C)])
        return 0
    lax.fori_loop(0, NCH, chunk_body, 0)


@functools.partial(jax.jit, static_argnames=())
def kernel(coords, idx, shape_code):
    mesh = plsc.VectorSubcoreMesh(core_axis_name="c", subcore_axis_name="s")
    run = pl.kernel(
        _sc_body,
        out_type=jax.ShapeDtypeStruct((B * P * C,), jnp.float32),
        mesh=mesh,
        scratch_types=[
            pltpu.VMEM((16,), jnp.int32),
            pltpu.VMEM((C, COLS), jnp.float32),
            pltpu.VMEM((COLS * TPAD,), jnp.float32),
            pltpu.VMEM((CB,), jnp.float32),
            pltpu.VMEM((CB,), jnp.int32),
            pltpu.VMEM((CB,), jnp.float32),
            pltpu.VMEM((CH * C,), jnp.float32),
        ],
        compiler_params=pltpu.CompilerParams(
            needs_layout_passes=False, use_tc_tiling_on_sc=True),
    )
    coords2 = coords.reshape(B * P * F)
    idxp = jnp.pad(idx.astype(jnp.int32), (0, 16 - B))
    out = run(coords2, idxp, shape_code)
    return out.reshape(B, P, C)


# speedup vs baseline: 13.5236x; 1.0183x over previous
import functools

import jax
import jax.numpy as jnp
from jax import lax
from jax.experimental import pallas as pl
from jax.experimental.pallas import tpu as pltpu
from jax.experimental.pallas import tpu_sc as plsc

B = 8
P = 20000
F = 3
C = 64
CODE_NUM = 256
COLS = CODE_NUM * F
TPAD = C + 1
NW = 32
PPW = P * B // NW
NCH = 25
CH = PPW // NCH
CB = CH * F + 24
NG = (CH * F + 8) // 16


def _sc_body(coords_hbm, idxp_hbm, code_hbm, out_hbm,
             idx_v, slice_v, t_v, coords_v, col_v, frac_v, out_v):
    cid = lax.axis_index("c")
    sid = lax.axis_index("s")
    wid = sid * 2 + cid
    b = wid // 4
    q = wid - b * 4

    lanes = lax.iota(jnp.int32, 16)

    pltpu.sync_copy(idxp_hbm, idx_v)
    myidx = plsc.load_gather(idx_v, [jnp.zeros((16,), jnp.int32) + b])[0]
    off = myidx * COLS
    pltpu.sync_copy(code_hbm.at[:, pl.ds(off, COLS)], slice_v)

    def tr_body(j, _):
        base = (j * 16 + lanes) * TPAD
        for c in range(C):
            v = slice_v[c, pl.ds(j * 16, 16)]
            plsc.store_scatter(t_v, [base + c], v)
        return 0
    lax.fori_loop(0, COLS // 16, tr_body, 0)

    def chunk_body(i, _):
        p0 = q * PPW + i * CH
        pltpu.sync_copy(coords_hbm.at[pl.ds((b * P + p0) * F, CH * F)],
                        coords_v.at[pl.ds(0, CH * F)])

        def pa_body(g, _):
            e0 = g * 16
            v = coords_v[pl.ds(e0, 16)]
            sc = (v + 1.0) * ((CODE_NUM - 1) / 2.0)
            c0 = sc.astype(jnp.int32)
            frac = sc - c0.astype(jnp.float32)
            f_lane = lax.rem(e0 + lanes, F)
            col = c0 + CODE_NUM * f_lane
            col_v[pl.ds(e0, 16)] = col
            frac_v[pl.ds(e0, 16)] = frac
            return 0
        lax.fori_loop(0, NG, pa_body, 0)

        def pb_body(p, _):
            p3 = p * F
            po = p * C
            colv = col_v[pl.ds(p3, 16)]
            fracv = frac_v[pl.ds(p3, 16)]
            accs = [None] * 4
            for f in range(F):
                cf = colv[f]
                tf = fracv[f]
                sf = 1.0 - tf
                base = cf * TPAD
                for k in range(4):
                    v0 = t_v[pl.ds(base + 16 * k, 16)]
                    v1 = t_v[pl.ds(base + TPAD + 16 * k, 16)]
                    term = v0 * sf + v1 * tf
                    accs[k] = term if accs[k] is None else accs[k] + term
            for k in range(4):
                out_v[pl.ds(po + 16 * k, 16)] = accs[k]
            return 0
        lax.fori_loop(0, CH, pb_body, 0, unroll=4)

        pltpu.sync_copy(out_v, out_hbm.at[pl.ds((b * P + p0) * C, CH * C)])
        return 0
    lax.fori_loop(0, NCH, chunk_body, 0)


@functools.partial(jax.jit, static_argnames=())
def kernel(coords, idx, shape_code):
    mesh = plsc.VectorSubcoreMesh(core_axis_name="c", subcore_axis_name="s")
    run = pl.kernel(
        _sc_body,
        out_type=jax.ShapeDtypeStruct((B * P * C,), jnp.float32),
        mesh=mesh,
        scratch_types=[
            pltpu.VMEM((16,), jnp.int32),
            pltpu.VMEM((C, COLS), jnp.float32),
            pltpu.VMEM((COLS * TPAD,), jnp.float32),
            pltpu.VMEM((CB,), jnp.float32),
            pltpu.VMEM((CB,), jnp.int32),
            pltpu.VMEM((CB,), jnp.float32),
            pltpu.VMEM((CH * C,), jnp.float32),
        ],
        compiler_params=pltpu.CompilerParams(
            needs_layout_passes=False, use_tc_tiling_on_sc=True),
    )
    coords2 = coords.reshape(B * P * F)
    idxp = jnp.pad(idx.astype(jnp.int32), (0, 16 - B))
    out = run(coords2, idxp, shape_code)
    return out.reshape(B, P, C)


# speedup vs baseline: 18.9491x; 1.4012x over previous
import functools

import jax
import jax.numpy as jnp
from jax import lax
from jax.experimental import pallas as pl
from jax.experimental.pallas import tpu as pltpu
from jax.experimental.pallas import tpu_sc as plsc

B = 8
P = 20000
F = 3
C = 64
CODE_NUM = 256
COLS = CODE_NUM * F
TPAD = C + 1
OPAD = 201
NW = 32
PPW = P // 4
NCH = 25
CH = PPW // NCH
CBUF = PPW + 8
IBUF = CH * F + 40


def _interp_chunk(i, q, lanes, cx_v, cy_v, cz_v, col_v, frac_v, t_v, out_x):
    p0l = i * CH

    def pa_body(g, _):
        e0 = p0l + g * 16
        tgt = (g * 16 + lanes) * F
        for f, cf_v in ((0, cx_v), (1, cy_v), (2, cz_v)):
            v = cf_v[pl.ds(e0, 16)]
            sc = (v + 1.0) * ((CODE_NUM - 1) / 2.0)
            c0 = sc.astype(jnp.int32)
            frac = sc - c0.astype(jnp.float32)
            plsc.store_scatter(col_v, [tgt + f], (c0 + CODE_NUM * f) * TPAD)
            plsc.store_scatter(frac_v, [tgt + f], frac)
        return 0
    lax.fori_loop(0, CH // 16 + 1, pa_body, 0)

    def pb_body(p, _):
        p3 = p * F
        po = p * C
        colv = col_v[pl.ds(p3, 16)]
        fracv = frac_v[pl.ds(p3, 16)]
        accs = [None] * 4
        for f in range(F):
            base = colv[f]
            tf = fracv[f]
            sf = 1.0 - tf
            for k in range(4):
                v0 = t_v[pl.ds(base + 16 * k, 16)]
                v1 = t_v[pl.ds(base + TPAD + 16 * k, 16)]
                term = v0 * sf + v1 * tf
                accs[k] = term if accs[k] is None else accs[k] + term
        for k in range(4):
            out_x[pl.ds(po + 16 * k, 16)] = accs[k]
        return 0
    lax.fori_loop(0, CH, pb_body, 0, unroll=2)


def _sc_body(c1_hbm, idxp_hbm, code_hbm, out_hbm,
             idx_v, stage8, t_v, cx_v, cy_v, cz_v, col_v, frac_v,
             out_a, out_b, sem):
    cid = lax.axis_index("c")
    sid = lax.axis_index("s")
    wid = sid * 2 + cid
    b = wid // 4
    q = wid - b * 4

    lanes = lax.iota(jnp.int32, 16)

    pltpu.sync_copy(idxp_hbm, idx_v)
    myidx = plsc.load_gather(idx_v, [jnp.zeros((16,), jnp.int32) + b])[0]
    off = myidx * COLS
    for r0 in range(0, C, 8):
        pltpu.sync_copy(code_hbm.at[pl.ds(r0, 8), pl.ds(off, COLS)], stage8)

        def tr_body(j, _):
            jv = (j * 16 + lanes) * TPAD + r0
            for rr in range(8):
                v = stage8[rr, pl.ds(j * 16, 16)]
                plsc.store_scatter(t_v, [jv + rr], v)
            return 0
        lax.fori_loop(0, COLS // 16, tr_body, 0)

    base_w = b * P + q * PPW
    for f, cf_v in ((0, cx_v), (1, cy_v), (2, cz_v)):
        pltpu.sync_copy(c1_hbm.at[pl.ds(f * B * P + base_w, PPW)],
                        cf_v.at[pl.ds(0, PPW)])

    ch_args = (q, lanes, cx_v, cy_v, cz_v, col_v, frac_v, t_v)
    obase = (b * P + q * PPW) * C

    def fire(out_x, i):
        pltpu.make_async_copy(
            out_x, out_hbm.at[pl.ds(obase + i * CH * C, CH * C)], sem).start()

    def wait_one(out_x):
        pltpu.make_async_copy(
            out_hbm.at[pl.ds(0, CH * C)], out_x, sem).wait()

    def pair_body(ii, _):
        i0 = ii * 2

        @pl.when(ii >= 1)
        def _():
            wait_one(out_a)
        _interp_chunk(i0, *ch_args, out_a)
        fire(out_a, i0)

        @pl.when(ii >= 1)
        def _():
            wait_one(out_b)
        _interp_chunk(i0 + 1, *ch_args, out_b)
        fire(out_b, i0 + 1)
        return 0
    lax.fori_loop(0, (NCH - 1) // 2, pair_body, 0)

    wait_one(out_a)
    _interp_chunk(NCH - 1, *ch_args, out_a)
    fire(out_a, NCH - 1)
    wait_one(out_b)
    wait_one(out_a)


@functools.partial(jax.jit, static_argnames=())
def kernel(coords, idx, shape_code):
    mesh = plsc.VectorSubcoreMesh(core_axis_name="c", subcore_axis_name="s")
    run = pl.kernel(
        _sc_body,
        out_type=jax.ShapeDtypeStruct((B * P * C,), jnp.float32),
        mesh=mesh,
        scratch_types=[
            pltpu.VMEM((16,), jnp.int32),
            pltpu.VMEM((8, COLS), jnp.float32),
            pltpu.VMEM((COLS * TPAD,), jnp.float32),
            pltpu.VMEM((CBUF,), jnp.float32),
            pltpu.VMEM((CBUF,), jnp.float32),
            pltpu.VMEM((CBUF,), jnp.float32),
            pltpu.VMEM((IBUF,), jnp.int32),
            pltpu.VMEM((IBUF,), jnp.float32),
            pltpu.VMEM((CH * C,), jnp.float32),
            pltpu.VMEM((CH * C,), jnp.float32),
            pltpu.SemaphoreType.DMA,
        ],
        compiler_params=pltpu.CompilerParams(needs_layout_passes=False),
    )
    c1 = coords.transpose(2, 0, 1).reshape(B * P * F)
    idxp = jnp.pad(idx.astype(jnp.int32), (0, 16 - B))
    out = run(c1, idxp, shape_code)
    return out.reshape(B, P, C)
